# initial kernel scaffold (unmeasured)
import jax
import jax.numpy as jnp
from jax import lax
from jax.experimental import pallas as pl
from jax.experimental.pallas import tpu as pltpu

N_DEV = 8
EPD = 4
N_EXP = N_DEV * EPD
CAP = 104
D = 1024
F = 2048
BLK = EPD * CAP


def _body(s_ref, w1_ref, w2_ref, out_ref,
          r2, obig, w1v, w2v,
          local_sems, p1_send, p1_recv, p2_send, p2_recv):
    my = lax.axis_index("i")

    bsem = pltpu.get_barrier_semaphore()
    for o in range(1, N_DEV):
        pl.semaphore_signal(
            bsem, inc=1,
            device_id=(lax.rem(my + o, N_DEV),),
            device_id_type=pl.DeviceIdType.MESH,
        )
    pl.semaphore_wait(bsem, N_DEV - 1)

    p1_descs = []
    for o in range(1, N_DEV):
        dst = lax.rem(my + o, N_DEV)
        d = pltpu.make_async_remote_copy(
            src_ref=s_ref.at[pl.ds(dst * BLK, BLK)],
            dst_ref=r2.at[pl.ds(my * BLK, BLK)],
            send_sem=p1_send.at[o],
            recv_sem=p1_recv.at[o],
            device_id=(dst,),
            device_id_type=pl.DeviceIdType.MESH,
        )
        d.start()
        p1_descs.append(d)

    cp = pltpu.make_async_copy(
        s_ref.at[pl.ds(my * BLK, BLK)],
        r2.at[pl.ds(my * BLK, BLK)],
        local_sems.at[0],
    )
    cp.start()
    cp.wait()

    for o in range(1, N_DEV):
        src = lax.rem(my - o + N_DEV, N_DEV)
        rd = pltpu.make_async_remote_copy(
            src_ref=s_ref.at[pl.ds(0, BLK)],
            dst_ref=r2.at[pl.ds(src * BLK, BLK)],
            send_sem=p1_send.at[o],
            recv_sem=p1_recv.at[o],
            device_id=(src,),
            device_id_type=pl.DeviceIdType.MESH,
        )
        rd.wait_recv()

    for j in range(EPD):
        wc1 = pltpu.make_async_copy(w1_ref.at[j], w1v, local_sems.at[1])
        wc2 = pltpu.make_async_copy(w2_ref.at[j], w2v, local_sems.at[2])
        wc1.start()
        wc2.start()
        wc1.wait()
        wc2.wait()
        rj = jnp.concatenate(
            [r2[pl.ds((s * EPD + j) * CAP, CAP), :] for s in range(N_DEV)],
            axis=0,
        )
        h = jnp.maximum(
            jnp.dot(rj, w1v[...], preferred_element_type=jnp.float32), 0.0
        )
        oj = jnp.dot(h, w2v[...], preferred_element_type=jnp.float32)
        for s in range(N_DEV):
            obig[pl.ds((s * EPD + j) * CAP, CAP), :] = oj[s * CAP:(s + 1) * CAP, :]

    p2_descs = []
    for o in range(1, N_DEV):
        dst = lax.rem(my + o, N_DEV)
        d = pltpu.make_async_remote_copy(
            src_ref=obig.at[pl.ds(dst * BLK, BLK)],
            dst_ref=out_ref.at[pl.ds(my * BLK, BLK)],
            send_sem=p2_send.at[o],
            recv_sem=p2_recv.at[o],
            device_id=(dst,),
            device_id_type=pl.DeviceIdType.MESH,
        )
        d.start()
        p2_descs.append(d)

    cp2 = pltpu.make_async_copy(
        obig.at[pl.ds(my * BLK, BLK)],
        out_ref.at[pl.ds(my * BLK, BLK)],
        local_sems.at[0],
    )
    cp2.start()
    cp2.wait()

    for o in range(1, N_DEV):
        src = lax.rem(my - o + N_DEV, N_DEV)
        rd = pltpu.make_async_remote_copy(
            src_ref=obig.at[pl.ds(0, BLK)],
            dst_ref=out_ref.at[pl.ds(src * BLK, BLK)],
            send_sem=p2_send.at[o],
            recv_sem=p2_recv.at[o],
            device_id=(src,),
            device_id_type=pl.DeviceIdType.MESH,
        )
        rd.wait_recv()

    for d in p1_descs:
        d.wait_send()
    for d in p2_descs:
        d.wait_send()


def kernel(x, assign, W1, W2):
    t, d = x.shape
    assign = assign.astype(jnp.int32)

    onehot = (assign[:, None]
              == jnp.arange(N_EXP, dtype=jnp.int32)[None, :]).astype(jnp.int32)
    rank = jnp.sum(onehot * (jnp.cumsum(onehot, axis=0) - 1), axis=1)
    slot = assign * CAP + rank
    slot = jnp.where(rank < CAP, slot, N_EXP * CAP)

    s_buf = jnp.zeros((N_EXP * CAP, d), jnp.float32).at[slot].set(
        x, mode="drop"
    )

    ret = pl.pallas_call(
        _body,
        out_shape=jax.ShapeDtypeStruct((N_EXP * CAP, d), jnp.float32),
        in_specs=[
            pl.BlockSpec(memory_space=pltpu.ANY),
            pl.BlockSpec(memory_space=pltpu.ANY),
            pl.BlockSpec(memory_space=pltpu.ANY),
        ],
        out_specs=pl.BlockSpec(memory_space=pltpu.ANY),
        scratch_shapes=[
            pltpu.VMEM((N_DEV * BLK, D), jnp.float32),
            pltpu.VMEM((N_DEV * BLK, D), jnp.float32),
            pltpu.VMEM((D, F), jnp.float32),
            pltpu.VMEM((F, D), jnp.float32),
            pltpu.SemaphoreType.DMA((4,)),
            pltpu.SemaphoreType.DMA((N_DEV,)),
            pltpu.SemaphoreType.DMA((N_DEV,)),
            pltpu.SemaphoreType.DMA((N_DEV,)),
            pltpu.SemaphoreType.DMA((N_DEV,)),
        ],
        compiler_params=pltpu.CompilerParams(collective_id=0),
    )(s_buf, W1, W2)

    return jnp.take(ret, jnp.clip(slot, 0, N_EXP * CAP - 1), axis=0)


# baseline (device time: 421369 ns/iter reference)
import jax
import jax.numpy as jnp
from jax import lax
from jax.experimental import pallas as pl
from jax.experimental.pallas import tpu as pltpu

N_DEV = 8
EPD = 4
N_EXP = N_DEV * EPD
CAP = 104
D = 1024
F = 2048
BLK = EPD * CAP


def _body(s_ref, w1_ref, w2_ref, out_ref,
          r2, obig, w1v, w2v,
          local_sems, p1_send, p1_recv, p2_send, p2_recv):
    my = lax.axis_index("i")

    bsem = pltpu.get_barrier_semaphore()
    for o in range(1, N_DEV):
        pl.semaphore_signal(
            bsem, inc=1,
            device_id=(lax.rem(my + o, N_DEV),),
            device_id_type=pl.DeviceIdType.MESH,
        )
    pl.semaphore_wait(bsem, N_DEV - 1)

    p1_descs = []
    for o in range(1, N_DEV):
        dst = lax.rem(my + o, N_DEV)
        d = pltpu.make_async_remote_copy(
            src_ref=s_ref.at[pl.ds(dst * BLK, BLK)],
            dst_ref=r2.at[pl.ds(my * BLK, BLK)],
            send_sem=p1_send.at[o],
            recv_sem=p1_recv.at[o],
            device_id=(dst,),
            device_id_type=pl.DeviceIdType.MESH,
        )
        d.start()
        p1_descs.append(d)

    cp = pltpu.make_async_copy(
        s_ref.at[pl.ds(my * BLK, BLK)],
        r2.at[pl.ds(my * BLK, BLK)],
        local_sems.at[0],
    )
    cp.start()
    cp.wait()

    for o in range(1, N_DEV):
        src = lax.rem(my - o + N_DEV, N_DEV)
        rd = pltpu.make_async_remote_copy(
            src_ref=s_ref.at[pl.ds(0, BLK)],
            dst_ref=r2.at[pl.ds(src * BLK, BLK)],
            send_sem=p1_send.at[o],
            recv_sem=p1_recv.at[o],
            device_id=(src,),
            device_id_type=pl.DeviceIdType.MESH,
        )
        rd.wait_recv()

    for j in range(EPD):
        wc1 = pltpu.make_async_copy(w1_ref.at[j], w1v, local_sems.at[1])
        wc2 = pltpu.make_async_copy(w2_ref.at[j], w2v, local_sems.at[2])
        wc1.start()
        wc2.start()
        wc1.wait()
        wc2.wait()
        rj = jnp.concatenate(
            [r2[pl.ds((s * EPD + j) * CAP, CAP), :] for s in range(N_DEV)],
            axis=0,
        )
        h = jnp.maximum(
            jnp.dot(rj, w1v[...], preferred_element_type=jnp.float32), 0.0
        )
        oj = jnp.dot(h, w2v[...], preferred_element_type=jnp.float32)
        for s in range(N_DEV):
            obig[pl.ds((s * EPD + j) * CAP, CAP), :] = oj[s * CAP:(s + 1) * CAP, :]

    p2_descs = []
    for o in range(1, N_DEV):
        dst = lax.rem(my + o, N_DEV)
        d = pltpu.make_async_remote_copy(
            src_ref=obig.at[pl.ds(dst * BLK, BLK)],
            dst_ref=out_ref.at[pl.ds(my * BLK, BLK)],
            send_sem=p2_send.at[o],
            recv_sem=p2_recv.at[o],
            device_id=(dst,),
            device_id_type=pl.DeviceIdType.MESH,
        )
        d.start()
        p2_descs.append(d)

    cp2 = pltpu.make_async_copy(
        obig.at[pl.ds(my * BLK, BLK)],
        out_ref.at[pl.ds(my * BLK, BLK)],
        local_sems.at[0],
    )
    cp2.start()
    cp2.wait()

    for o in range(1, N_DEV):
        src = lax.rem(my - o + N_DEV, N_DEV)
        rd = pltpu.make_async_remote_copy(
            src_ref=obig.at[pl.ds(0, BLK)],
            dst_ref=out_ref.at[pl.ds(src * BLK, BLK)],
            send_sem=p2_send.at[o],
            recv_sem=p2_recv.at[o],
            device_id=(src,),
            device_id_type=pl.DeviceIdType.MESH,
        )
        rd.wait_recv()

    for d in p1_descs:
        d.wait_send()
    for d in p2_descs:
        d.wait_send()


def kernel(x, assign, W1, W2):
    t, d = x.shape
    assign = assign.astype(jnp.int32)

    onehot = (assign[:, None]
              == jnp.arange(N_EXP, dtype=jnp.int32)[None, :]).astype(jnp.int32)
    rank = jnp.sum(onehot * (jnp.cumsum(onehot, axis=0) - 1), axis=1)
    slot = assign * CAP + rank
    slot = jnp.where(rank < CAP, slot, N_EXP * CAP)

    s_buf = jnp.zeros((N_EXP * CAP, d), jnp.float32).at[slot].set(
        x, mode="drop"
    )

    ret = pl.pallas_call(
        _body,
        out_shape=jax.ShapeDtypeStruct((N_EXP * CAP, d), jnp.float32),
        in_specs=[
            pl.BlockSpec(memory_space=pl.ANY),
            pl.BlockSpec(memory_space=pl.ANY),
            pl.BlockSpec(memory_space=pl.ANY),
        ],
        out_specs=pl.BlockSpec(memory_space=pl.ANY),
        scratch_shapes=[
            pltpu.VMEM((N_DEV * BLK, D), jnp.float32),
            pltpu.VMEM((N_DEV * BLK, D), jnp.float32),
            pltpu.VMEM((D, F), jnp.float32),
            pltpu.VMEM((F, D), jnp.float32),
            pltpu.SemaphoreType.DMA((4,)),
            pltpu.SemaphoreType.DMA((N_DEV,)),
            pltpu.SemaphoreType.DMA((N_DEV,)),
            pltpu.SemaphoreType.DMA((N_DEV,)),
            pltpu.SemaphoreType.DMA((N_DEV,)),
        ],
        compiler_params=pltpu.CompilerParams(
            collective_id=0, vmem_limit_bytes=100 * 1024 * 1024
        ),
    )(s_buf, W1, W2)

    return jnp.take(ret, jnp.clip(slot, 0, N_EXP * CAP - 1), axis=0)


# device time: 305476 ns/iter; 1.3794x vs baseline; 1.3794x over previous
import jax
import jax.numpy as jnp
from jax import lax
from jax.experimental import pallas as pl
from jax.experimental.pallas import tpu as pltpu

N_DEV = 8
EPD = 4
N_EXP = N_DEV * EPD
CAP = 104
D = 1024
F = 2048
BLK = EPD * CAP


def _body(s_ref, w1_ref, w2_ref, out_ref,
          r2, obig, w1v, w2v,
          local_sems, p1_send, p1_recv, p2_send, p2_recv):
    my = lax.axis_index("i")

    bsem = pltpu.get_barrier_semaphore()
    for o in range(1, N_DEV):
        pl.semaphore_signal(
            bsem, inc=1,
            device_id=(lax.rem(my + o, N_DEV),),
            device_id_type=pl.DeviceIdType.MESH,
        )
    pl.semaphore_wait(bsem, N_DEV - 1)

    p1_descs = []
    for o in range(1, N_DEV):
        dst = lax.rem(my + o, N_DEV)
        d = pltpu.make_async_remote_copy(
            src_ref=s_ref.at[pl.ds(dst * BLK, BLK)],
            dst_ref=r2.at[pl.ds(my * BLK, BLK)],
            send_sem=p1_send.at[o],
            recv_sem=p1_recv.at[o],
            device_id=(dst,),
            device_id_type=pl.DeviceIdType.MESH,
        )
        d.start()
        p1_descs.append(d)

    cp = pltpu.make_async_copy(
        s_ref.at[pl.ds(my * BLK, BLK)],
        r2.at[pl.ds(my * BLK, BLK)],
        local_sems.at[0],
    )
    cp.start()
    cp.wait()

    for o in range(1, N_DEV):
        src = lax.rem(my - o + N_DEV, N_DEV)
        rd = pltpu.make_async_remote_copy(
            src_ref=s_ref.at[pl.ds(0, BLK)],
            dst_ref=r2.at[pl.ds(src * BLK, BLK)],
            send_sem=p1_send.at[o],
            recv_sem=p1_recv.at[o],
            device_id=(src,),
            device_id_type=pl.DeviceIdType.MESH,
        )
        rd.wait_recv()

    for j in range(EPD):
        wc1 = pltpu.make_async_copy(w1_ref.at[j], w1v, local_sems.at[1])
        wc2 = pltpu.make_async_copy(w2_ref.at[j], w2v, local_sems.at[2])
        wc1.start()
        wc2.start()
        wc1.wait()
        wc2.wait()
        rj = jnp.concatenate(
            [r2[pl.ds((s * EPD + j) * CAP, CAP), :] for s in range(N_DEV)],
            axis=0,
        )
        h = jnp.maximum(
            jnp.dot(rj, w1v[...], preferred_element_type=jnp.float32), 0.0
        )
        oj = jnp.dot(h, w2v[...], preferred_element_type=jnp.float32)
        for s in range(N_DEV):
            obig[pl.ds((s * EPD + j) * CAP, CAP), :] = oj[s * CAP:(s + 1) * CAP, :]

    p2_descs = []
    for o in range(1, N_DEV):
        dst = lax.rem(my + o, N_DEV)
        d = pltpu.make_async_remote_copy(
            src_ref=obig.at[pl.ds(dst * BLK, BLK)],
            dst_ref=out_ref.at[pl.ds(my * BLK, BLK)],
            send_sem=p2_send.at[o],
            recv_sem=p2_recv.at[o],
            device_id=(dst,),
            device_id_type=pl.DeviceIdType.MESH,
        )
        d.start()
        p2_descs.append(d)

    cp2 = pltpu.make_async_copy(
        obig.at[pl.ds(my * BLK, BLK)],
        out_ref.at[pl.ds(my * BLK, BLK)],
        local_sems.at[0],
    )
    cp2.start()
    cp2.wait()

    for o in range(1, N_DEV):
        src = lax.rem(my - o + N_DEV, N_DEV)
        rd = pltpu.make_async_remote_copy(
            src_ref=obig.at[pl.ds(0, BLK)],
            dst_ref=out_ref.at[pl.ds(src * BLK, BLK)],
            send_sem=p2_send.at[o],
            recv_sem=p2_recv.at[o],
            device_id=(src,),
            device_id_type=pl.DeviceIdType.MESH,
        )
        rd.wait_recv()

    for d in p1_descs:
        d.wait_send()
    for d in p2_descs:
        d.wait_send()


def kernel(x, assign, W1, W2):
    t, d = x.shape
    assign = assign.astype(jnp.int32)

    onehot = (assign[:, None]
              == jnp.arange(N_EXP, dtype=jnp.int32)[None, :]).astype(jnp.int32)
    rank = jnp.sum(onehot * (jnp.cumsum(onehot, axis=0) - 1), axis=1)
    slot = assign * CAP + rank
    slot = jnp.where(rank < CAP, slot, N_EXP * CAP)

    s_buf = jnp.zeros((N_EXP * CAP, d), jnp.float32).at[slot].set(
        x, mode="drop"
    )

    ret = pl.pallas_call(
        _body,
        out_shape=jax.ShapeDtypeStruct((N_EXP * CAP, d), jnp.float32),
        in_specs=[
            pl.BlockSpec(memory_space=pl.ANY),
            pl.BlockSpec(memory_space=pl.ANY),
            pl.BlockSpec(memory_space=pl.ANY),
        ],
        out_specs=pl.BlockSpec(memory_space=pl.ANY),
        scratch_shapes=[
            pltpu.VMEM((N_DEV * BLK, D), jnp.float32),
            pltpu.VMEM((N_DEV * BLK, D), jnp.float32),
            pltpu.VMEM((D, F), jnp.float32),
            pltpu.VMEM((F, D), jnp.float32),
            pltpu.SemaphoreType.DMA((4,)),
            pltpu.SemaphoreType.DMA((N_DEV,)),
            pltpu.SemaphoreType.DMA((N_DEV,)),
            pltpu.SemaphoreType.DMA((N_DEV,)),
            pltpu.SemaphoreType.DMA((N_DEV,)),
        ],
        compiler_params=pltpu.CompilerParams(
            collective_id=0, vmem_limit_bytes=100 * 1024 * 1024
        ),
    )(s_buf, W1, W2)

    inv = jnp.full((N_EXP * CAP,), t, jnp.int32).at[slot].set(
        jnp.arange(t, dtype=jnp.int32), mode="drop"
    )
    return jnp.zeros((t, d), jnp.float32).at[inv].set(ret, mode="drop")


# device time: 219364 ns/iter; 1.9209x vs baseline; 1.3926x over previous
import jax
import jax.numpy as jnp
from jax import lax
from jax.experimental import pallas as pl
from jax.experimental.pallas import tpu as pltpu

N_DEV = 8
EPD = 4
N_EXP = N_DEV * EPD
CAP = 96
D = 1024
F = 2048
BLK = EPD * CAP


def _body(s_ref, w1_ref, w2_ref, out_ref,
          r2, obig, w1v, w2v,
          local_sems, p1_send, p1_recv, p2_send, p2_recv):
    my = lax.axis_index("i")

    bsem = pltpu.get_barrier_semaphore()
    for o in range(1, N_DEV):
        pl.semaphore_signal(
            bsem, inc=1,
            device_id=(lax.rem(my + o, N_DEV),),
            device_id_type=pl.DeviceIdType.MESH,
        )
    pl.semaphore_wait(bsem, N_DEV - 1)

    p1_descs = []
    for j in range(EPD):
        for o in range(1, N_DEV):
            dst = lax.rem(my + o, N_DEV)
            d = pltpu.make_async_remote_copy(
                src_ref=s_ref.at[pl.ds((dst * EPD + j) * CAP, CAP)],
                dst_ref=r2.at[pl.ds((my * EPD + j) * CAP, CAP)],
                send_sem=p1_send.at[j, o],
                recv_sem=p1_recv.at[j, o],
                device_id=(dst,),
                device_id_type=pl.DeviceIdType.MESH,
            )
            d.start()
            p1_descs.append(d)

    cp = pltpu.make_async_copy(
        s_ref.at[pl.ds(my * BLK, BLK)],
        r2.at[pl.ds(my * BLK, BLK)],
        local_sems.at[0],
    )
    cp.start()

    wc1 = pltpu.make_async_copy(w1_ref.at[0], w1v.at[0], local_sems.at[1])
    wc2 = pltpu.make_async_copy(w2_ref.at[0], w2v.at[0], local_sems.at[2])
    wc1.start()
    wc2.start()
    cp.wait()

    p2_descs = []
    for j in range(EPD):
        pltpu.make_async_copy(w1_ref.at[j], w1v.at[j % 2],
                              local_sems.at[1]).wait()
        pltpu.make_async_copy(w2_ref.at[j], w2v.at[j % 2],
                              local_sems.at[2]).wait()
        if j + 1 < EPD:
            nw1 = pltpu.make_async_copy(w1_ref.at[j + 1], w1v.at[(j + 1) % 2],
                                        local_sems.at[1])
            nw2 = pltpu.make_async_copy(w2_ref.at[j + 1], w2v.at[(j + 1) % 2],
                                        local_sems.at[2])
            nw1.start()
            nw2.start()

        for o in range(1, N_DEV):
            src = lax.rem(my - o + N_DEV, N_DEV)
            rd = pltpu.make_async_remote_copy(
                src_ref=s_ref.at[pl.ds(0, CAP)],
                dst_ref=r2.at[pl.ds((src * EPD + j) * CAP, CAP)],
                send_sem=p1_send.at[j, o],
                recv_sem=p1_recv.at[j, o],
                device_id=(src,),
                device_id_type=pl.DeviceIdType.MESH,
            )
            rd.wait_recv()

        rj = jnp.concatenate(
            [r2[pl.ds((s * EPD + j) * CAP, CAP), :] for s in range(N_DEV)],
            axis=0,
        )
        h = jnp.maximum(
            jnp.dot(rj, w1v[j % 2], preferred_element_type=jnp.float32), 0.0
        )
        oj = jnp.dot(h, w2v[j % 2], preferred_element_type=jnp.float32)
        for s in range(N_DEV):
            obig[pl.ds((s * EPD + j) * CAP, CAP), :] = oj[s * CAP:(s + 1) * CAP, :]

        for o in range(1, N_DEV):
            dst = lax.rem(my + o, N_DEV)
            d = pltpu.make_async_remote_copy(
                src_ref=obig.at[pl.ds((dst * EPD + j) * CAP, CAP)],
                dst_ref=out_ref.at[pl.ds((my * EPD + j) * CAP, CAP)],
                send_sem=p2_send.at[j, o],
                recv_sem=p2_recv.at[j, o],
                device_id=(dst,),
                device_id_type=pl.DeviceIdType.MESH,
            )
            d.start()
            p2_descs.append(d)
        cpo = pltpu.make_async_copy(
            obig.at[pl.ds((my * EPD + j) * CAP, CAP)],
            out_ref.at[pl.ds((my * EPD + j) * CAP, CAP)],
            local_sems.at[3],
        )
        cpo.start()
        cpo.wait()

    for j in range(EPD):
        for o in range(1, N_DEV):
            src = lax.rem(my - o + N_DEV, N_DEV)
            rd = pltpu.make_async_remote_copy(
                src_ref=obig.at[pl.ds(0, CAP)],
                dst_ref=out_ref.at[pl.ds((src * EPD + j) * CAP, CAP)],
                send_sem=p2_send.at[j, o],
                recv_sem=p2_recv.at[j, o],
                device_id=(src,),
                device_id_type=pl.DeviceIdType.MESH,
            )
            rd.wait_recv()

    for d in p1_descs:
        d.wait_send()
    for d in p2_descs:
        d.wait_send()


def kernel(x, assign, W1, W2):
    t, d = x.shape
    assign = assign.astype(jnp.int32)

    onehot = (assign[:, None]
              == jnp.arange(N_EXP, dtype=jnp.int32)[None, :]).astype(jnp.int32)
    rank = jnp.sum(onehot * (jnp.cumsum(onehot, axis=0) - 1), axis=1)
    slot = assign * CAP + rank
    slot = jnp.where(rank < CAP, slot, N_EXP * CAP)

    s_buf = jnp.zeros((N_EXP * CAP, d), jnp.float32).at[slot].set(
        x, mode="drop"
    )

    ret = pl.pallas_call(
        _body,
        out_shape=jax.ShapeDtypeStruct((N_EXP * CAP, d), jnp.float32),
        in_specs=[
            pl.BlockSpec(memory_space=pl.ANY),
            pl.BlockSpec(memory_space=pl.ANY),
            pl.BlockSpec(memory_space=pl.ANY),
        ],
        out_specs=pl.BlockSpec(memory_space=pl.ANY),
        scratch_shapes=[
            pltpu.VMEM((N_DEV * BLK, D), jnp.float32),
            pltpu.VMEM((N_DEV * BLK, D), jnp.float32),
            pltpu.VMEM((2, D, F), jnp.float32),
            pltpu.VMEM((2, F, D), jnp.float32),
            pltpu.SemaphoreType.DMA((4,)),
            pltpu.SemaphoreType.DMA((EPD, N_DEV)),
            pltpu.SemaphoreType.DMA((EPD, N_DEV)),
            pltpu.SemaphoreType.DMA((EPD, N_DEV)),
            pltpu.SemaphoreType.DMA((EPD, N_DEV)),
        ],
        compiler_params=pltpu.CompilerParams(
            collective_id=0, vmem_limit_bytes=100 * 1024 * 1024
        ),
    )(s_buf, W1, W2)

    inv = jnp.full((N_EXP * CAP,), t, jnp.int32).at[slot].set(
        jnp.arange(t, dtype=jnp.int32), mode="drop"
    )
    return jnp.zeros((t, d), jnp.float32).at[inv].set(ret, mode="drop")


# device time: 167596 ns/iter; 2.5142x vs baseline; 1.3089x over previous
import jax
import jax.numpy as jnp
from jax import lax
from jax.experimental import pallas as pl
from jax.experimental.pallas import tpu as pltpu

N_DEV = 8
EPD = 4
N_EXP = N_DEV * EPD
CAP = 96
D = 1024
F = 2048
BLK = EPD * CAP
ROWS = N_DEV * CAP


def _body(s_ref, w1_ref, w2_ref, out_ref,
          r2, obig, w1v, w2v,
          local_sems, p1_send, p1_recv, p2_send, p2_recv):
    my = lax.axis_index("i")

    bsem = pltpu.get_barrier_semaphore()
    for o in range(1, N_DEV):
        pl.semaphore_signal(
            bsem, inc=1,
            device_id=(lax.rem(my + o, N_DEV),),
            device_id_type=pl.DeviceIdType.MESH,
        )
    pl.semaphore_wait(bsem, N_DEV - 1)

    p1_descs = []
    for j in range(EPD):
        for o in range(1, N_DEV):
            dst = lax.rem(my + o, N_DEV)
            d = pltpu.make_async_remote_copy(
                src_ref=s_ref.at[pl.ds((dst * EPD + j) * CAP, CAP)],
                dst_ref=r2.at[pl.ds(j * ROWS + my * CAP, CAP)],
                send_sem=p1_send.at[j, o],
                recv_sem=p1_recv.at[j, o],
                device_id=(dst,),
                device_id_type=pl.DeviceIdType.MESH,
            )
            d.start()
            p1_descs.append(d)

    for j in range(EPD):
        cp = pltpu.make_async_copy(
            s_ref.at[pl.ds((my * EPD + j) * CAP, CAP)],
            r2.at[pl.ds(j * ROWS + my * CAP, CAP)],
            local_sems.at[j],
        )
        cp.start()

    pltpu.make_async_copy(w1_ref.at[0], w1v.at[0], local_sems.at[4]).start()
    pltpu.make_async_copy(w2_ref.at[0], w2v.at[0], local_sems.at[5]).start()

    p2_descs = []
    for j in range(EPD):
        pltpu.make_async_copy(w1_ref.at[j], w1v.at[j % 2],
                              local_sems.at[4]).wait()
        pltpu.make_async_copy(w2_ref.at[j], w2v.at[j % 2],
                              local_sems.at[5]).wait()
        if j + 1 < EPD:
            pltpu.make_async_copy(w1_ref.at[j + 1], w1v.at[(j + 1) % 2],
                                  local_sems.at[4]).start()
            pltpu.make_async_copy(w2_ref.at[j + 1], w2v.at[(j + 1) % 2],
                                  local_sems.at[5]).start()

        pltpu.make_async_copy(
            s_ref.at[pl.ds((my * EPD + j) * CAP, CAP)],
            r2.at[pl.ds(j * ROWS + my * CAP, CAP)],
            local_sems.at[j],
        ).wait()
        for o in range(1, N_DEV):
            src = lax.rem(my - o + N_DEV, N_DEV)
            rd = pltpu.make_async_remote_copy(
                src_ref=s_ref.at[pl.ds(0, CAP)],
                dst_ref=r2.at[pl.ds(j * ROWS + src * CAP, CAP)],
                send_sem=p1_send.at[j, o],
                recv_sem=p1_recv.at[j, o],
                device_id=(src,),
                device_id_type=pl.DeviceIdType.MESH,
            )
            rd.wait_recv()

        rj = r2[pl.ds(j * ROWS, ROWS), :]
        h = jnp.dot(rj, w1v[j % 2], preferred_element_type=jnp.float32)
        h = jnp.maximum(h, 0.0).astype(jnp.bfloat16)
        oj = jnp.dot(h, w2v[j % 2], preferred_element_type=jnp.float32)
        obig[pl.ds(j * ROWS, ROWS), :] = oj.astype(jnp.bfloat16)

        for o in range(1, N_DEV):
            dst = lax.rem(my + o, N_DEV)
            d = pltpu.make_async_remote_copy(
                src_ref=obig.at[pl.ds(j * ROWS + dst * CAP, CAP)],
                dst_ref=out_ref.at[pl.ds((my * EPD + j) * CAP, CAP)],
                send_sem=p2_send.at[j, o],
                recv_sem=p2_recv.at[j, o],
                device_id=(dst,),
                device_id_type=pl.DeviceIdType.MESH,
            )
            d.start()
            p2_descs.append(d)
        cpo = pltpu.make_async_copy(
            obig.at[pl.ds(j * ROWS + my * CAP, CAP)],
            out_ref.at[pl.ds((my * EPD + j) * CAP, CAP)],
            local_sems.at[j],
        )
        cpo.start()
        cpo.wait()

    for j in range(EPD):
        for o in range(1, N_DEV):
            src = lax.rem(my - o + N_DEV, N_DEV)
            rd = pltpu.make_async_remote_copy(
                src_ref=obig.at[pl.ds(0, CAP)],
                dst_ref=out_ref.at[pl.ds((src * EPD + j) * CAP, CAP)],
                send_sem=p2_send.at[j, o],
                recv_sem=p2_recv.at[j, o],
                device_id=(src,),
                device_id_type=pl.DeviceIdType.MESH,
            )
            rd.wait_recv()

    for d in p1_descs:
        d.wait_send()
    for d in p2_descs:
        d.wait_send()


def kernel(x, assign, W1, W2):
    t, d = x.shape
    assign = assign.astype(jnp.int32)

    onehot = (assign[:, None]
              == jnp.arange(N_EXP, dtype=jnp.int32)[None, :]).astype(jnp.int32)
    rank = jnp.sum(onehot * (jnp.cumsum(onehot, axis=0) - 1), axis=1)
    slot = assign * CAP + rank
    slot = jnp.where(rank < CAP, slot, N_EXP * CAP)

    s_buf = jnp.zeros((N_EXP * CAP, d), jnp.bfloat16).at[slot].set(
        x.astype(jnp.bfloat16), mode="drop"
    )

    ret = pl.pallas_call(
        _body,
        out_shape=jax.ShapeDtypeStruct((N_EXP * CAP, d), jnp.bfloat16),
        in_specs=[
            pl.BlockSpec(memory_space=pl.ANY),
            pl.BlockSpec(memory_space=pl.ANY),
            pl.BlockSpec(memory_space=pl.ANY),
        ],
        out_specs=pl.BlockSpec(memory_space=pl.ANY),
        scratch_shapes=[
            pltpu.VMEM((EPD * ROWS, D), jnp.bfloat16),
            pltpu.VMEM((EPD * ROWS, D), jnp.bfloat16),
            pltpu.VMEM((2, D, F), jnp.bfloat16),
            pltpu.VMEM((2, F, D), jnp.bfloat16),
            pltpu.SemaphoreType.DMA((6,)),
            pltpu.SemaphoreType.DMA((EPD, N_DEV)),
            pltpu.SemaphoreType.DMA((EPD, N_DEV)),
            pltpu.SemaphoreType.DMA((EPD, N_DEV)),
            pltpu.SemaphoreType.DMA((EPD, N_DEV)),
        ],
        compiler_params=pltpu.CompilerParams(
            collective_id=0, vmem_limit_bytes=100 * 1024 * 1024
        ),
    )(s_buf, W1.astype(jnp.bfloat16), W2.astype(jnp.bfloat16))

    inv = jnp.full((N_EXP * CAP,), t, jnp.int32).at[slot].set(
        jnp.arange(t, dtype=jnp.int32), mode="drop"
    )
    out = jnp.zeros((t, d), jnp.bfloat16).at[inv].set(ret, mode="drop")
    return out.astype(jnp.float32)


# device time: 135492 ns/iter; 3.1099x vs baseline; 1.2369x over previous
import jax
import jax.numpy as jnp
from jax import lax
from jax.experimental import pallas as pl
from jax.experimental.pallas import tpu as pltpu

N_DEV = 8
EPD = 4
N_EXP = N_DEV * EPD
CAP = 96
D = 1024
F = 2048
BLK = EPD * CAP
ROWS = N_DEV * CAP


def _body(s_ref, w1_ref, w2_ref, out_ref,
          r2, obig, w1f, w2f, w1v, w2v,
          local_sems, p1_send, p1_recv, p2_send, p2_recv):
    my = lax.axis_index("i")

    bsem = pltpu.get_barrier_semaphore()
    for o in range(1, N_DEV):
        pl.semaphore_signal(
            bsem, inc=1,
            device_id=(lax.rem(my + o, N_DEV),),
            device_id_type=pl.DeviceIdType.MESH,
        )
    pl.semaphore_wait(bsem, N_DEV - 1)

    p1_descs = []
    for j in range(EPD):
        for o in range(1, N_DEV):
            dst = lax.rem(my + o, N_DEV)
            d = pltpu.make_async_remote_copy(
                src_ref=s_ref.at[pl.ds((dst * EPD + j) * CAP, CAP)],
                dst_ref=r2.at[pl.ds(j * ROWS + my * CAP, CAP)],
                send_sem=p1_send.at[j, o],
                recv_sem=p1_recv.at[j, o],
                device_id=(dst,),
                device_id_type=pl.DeviceIdType.MESH,
            )
            d.start()
            p1_descs.append(d)

    for j in range(EPD):
        cp = pltpu.make_async_copy(
            s_ref.at[pl.ds((my * EPD + j) * CAP, CAP)],
            r2.at[pl.ds(j * ROWS + my * CAP, CAP)],
            local_sems.at[j],
        )
        cp.start()

    pltpu.make_async_copy(w1_ref.at[0], w1f.at[0], local_sems.at[4]).start()
    pltpu.make_async_copy(w2_ref.at[0], w2f.at[0], local_sems.at[5]).start()

    p2_descs = []
    for j in range(EPD):
        pltpu.make_async_copy(w1_ref.at[j], w1f.at[j % 2],
                              local_sems.at[4]).wait()
        pltpu.make_async_copy(w2_ref.at[j], w2f.at[j % 2],
                              local_sems.at[5]).wait()
        if j + 1 < EPD:
            pltpu.make_async_copy(w1_ref.at[j + 1], w1f.at[(j + 1) % 2],
                                  local_sems.at[4]).start()
            pltpu.make_async_copy(w2_ref.at[j + 1], w2f.at[(j + 1) % 2],
                                  local_sems.at[5]).start()
        w1v[j % 2] = w1f[j % 2].astype(jnp.bfloat16)
        w2v[j % 2] = w2f[j % 2].astype(jnp.bfloat16)

        pltpu.make_async_copy(
            s_ref.at[pl.ds((my * EPD + j) * CAP, CAP)],
            r2.at[pl.ds(j * ROWS + my * CAP, CAP)],
            local_sems.at[j],
        ).wait()
        for o in range(1, N_DEV):
            src = lax.rem(my - o + N_DEV, N_DEV)
            rd = pltpu.make_async_remote_copy(
                src_ref=s_ref.at[pl.ds(0, CAP)],
                dst_ref=r2.at[pl.ds(j * ROWS + src * CAP, CAP)],
                send_sem=p1_send.at[j, o],
                recv_sem=p1_recv.at[j, o],
                device_id=(src,),
                device_id_type=pl.DeviceIdType.MESH,
            )
            rd.wait_recv()

        rj = r2[pl.ds(j * ROWS, ROWS), :]
        h = jnp.dot(rj, w1v[j % 2], preferred_element_type=jnp.float32)
        h = jnp.maximum(h, 0.0).astype(jnp.bfloat16)
        oj = jnp.dot(h, w2v[j % 2], preferred_element_type=jnp.float32)
        obig[pl.ds(j * ROWS, ROWS), :] = oj.astype(jnp.bfloat16)

        for o in range(1, N_DEV):
            dst = lax.rem(my + o, N_DEV)
            d = pltpu.make_async_remote_copy(
                src_ref=obig.at[pl.ds(j * ROWS + dst * CAP, CAP)],
                dst_ref=out_ref.at[pl.ds((my * EPD + j) * CAP, CAP)],
                send_sem=p2_send.at[j, o],
                recv_sem=p2_recv.at[j, o],
                device_id=(dst,),
                device_id_type=pl.DeviceIdType.MESH,
            )
            d.start()
            p2_descs.append(d)
        cpo = pltpu.make_async_copy(
            obig.at[pl.ds(j * ROWS + my * CAP, CAP)],
            out_ref.at[pl.ds((my * EPD + j) * CAP, CAP)],
            local_sems.at[j],
        )
        cpo.start()
        cpo.wait()

    for j in range(EPD):
        for o in range(1, N_DEV):
            src = lax.rem(my - o + N_DEV, N_DEV)
            rd = pltpu.make_async_remote_copy(
                src_ref=obig.at[pl.ds(0, CAP)],
                dst_ref=out_ref.at[pl.ds((src * EPD + j) * CAP, CAP)],
                send_sem=p2_send.at[j, o],
                recv_sem=p2_recv.at[j, o],
                device_id=(src,),
                device_id_type=pl.DeviceIdType.MESH,
            )
            rd.wait_recv()

    for d in p1_descs:
        d.wait_send()
    for d in p2_descs:
        d.wait_send()


def kernel(x, assign, W1, W2):
    t, d = x.shape
    assign = assign.astype(jnp.int32)

    onehot = (assign[:, None]
              == jnp.arange(N_EXP, dtype=jnp.int32)[None, :]).astype(jnp.int32)
    rank = jnp.sum(onehot * (jnp.cumsum(onehot, axis=0) - 1), axis=1)
    slot = assign * CAP + rank
    slot = jnp.where(rank < CAP, slot, N_EXP * CAP)

    s_buf = jnp.zeros((N_EXP * CAP, d), jnp.bfloat16).at[slot].set(
        x.astype(jnp.bfloat16), mode="drop"
    )

    ret = pl.pallas_call(
        _body,
        out_shape=jax.ShapeDtypeStruct((N_EXP * CAP, d), jnp.bfloat16),
        in_specs=[
            pl.BlockSpec(memory_space=pl.ANY),
            pl.BlockSpec(memory_space=pl.ANY),
            pl.BlockSpec(memory_space=pl.ANY),
        ],
        out_specs=pl.BlockSpec(memory_space=pl.ANY),
        scratch_shapes=[
            pltpu.VMEM((EPD * ROWS, D), jnp.bfloat16),
            pltpu.VMEM((EPD * ROWS, D), jnp.bfloat16),
            pltpu.VMEM((2, D, F), jnp.float32),
            pltpu.VMEM((2, F, D), jnp.float32),
            pltpu.VMEM((2, D, F), jnp.bfloat16),
            pltpu.VMEM((2, F, D), jnp.bfloat16),
            pltpu.SemaphoreType.DMA((6,)),
            pltpu.SemaphoreType.DMA((EPD, N_DEV)),
            pltpu.SemaphoreType.DMA((EPD, N_DEV)),
            pltpu.SemaphoreType.DMA((EPD, N_DEV)),
            pltpu.SemaphoreType.DMA((EPD, N_DEV)),
        ],
        compiler_params=pltpu.CompilerParams(
            collective_id=0, vmem_limit_bytes=100 * 1024 * 1024
        ),
    )(s_buf, W1, W2)

    inv = jnp.full((N_EXP * CAP,), t, jnp.int32).at[slot].set(
        jnp.arange(t, dtype=jnp.int32), mode="drop"
    )
    out = jnp.zeros((t, d), jnp.bfloat16).at[inv].set(ret, mode="drop")
    return out.astype(jnp.float32)
